# baseline (speedup 1.0000x reference)
import functools

import jax
import jax.numpy as jnp
from jax import lax
from jax.experimental import pallas as pl
from jax.experimental.pallas import tpu as pltpu
from jax.experimental.pallas import tpu_sc as plsc

BATCH = 16384
EMBED_DIM = 10
ROW = 16
NC = 2
NS = 16
NW = NC * NS
B_PER_W = BATCH // NW
CHUNK = 128
NCHUNK = B_PER_W // CHUNK


def _gather_body(u_idx, m_idx, ut, mt, u_out, m_out, idx_u, idx_m, ru, rm, sem):
  wid = lax.axis_index("s") * NC + lax.axis_index("c")
  base = wid * B_PER_W
  pltpu.sync_copy(u_idx.at[wid], idx_u)
  pltpu.sync_copy(m_idx.at[wid], idx_m)
  copies = []
  for j in range(NCHUNK):
    sl = pl.ds(j * CHUNK, CHUNK)
    copies.append(pltpu.async_copy(ut.at[idx_u.at[j]], ru.at[sl], sem))
    copies.append(pltpu.async_copy(mt.at[idx_m.at[j]], rm.at[sl], sem))
  for c in copies:
    c.wait()
  pltpu.sync_copy(ru, u_out.at[pl.ds(base, B_PER_W)])
  pltpu.sync_copy(rm, m_out.at[pl.ds(base, B_PER_W)])


_sc_gather = functools.partial(
    pl.kernel,
    out_type=(
        jax.ShapeDtypeStruct((BATCH, ROW), jnp.float32),
        jax.ShapeDtypeStruct((BATCH, ROW), jnp.float32),
    ),
    mesh=plsc.VectorSubcoreMesh(core_axis_name="c", subcore_axis_name="s"),
    scratch_types=[
        pltpu.VMEM((NCHUNK, CHUNK), jnp.int32),
        pltpu.VMEM((NCHUNK, CHUNK), jnp.int32),
        pltpu.VMEM((B_PER_W, ROW), jnp.float32),
        pltpu.VMEM((B_PER_W, ROW), jnp.float32),
        pltpu.SemaphoreType.DMA,
    ],
    compiler_params=pltpu.CompilerParams(use_tc_tiling_on_sc=False),
)(_gather_body)


def _mlp_body(u_ref, m_ref, w1u_ref, w1m_ref, b1_ref, w2_ref, b2_ref, o_ref):
  h = (
      jnp.dot(u_ref[...], w1u_ref[...], preferred_element_type=jnp.float32)
      + jnp.dot(m_ref[...], w1m_ref[...], preferred_element_type=jnp.float32)
      + b1_ref[...]
  )
  h = jnp.maximum(h, 0.0)
  o_ref[...] = (
      jnp.dot(h, w2_ref[...], preferred_element_type=jnp.float32) + b2_ref[...]
  )


def _mlp(u_rows, m_rows, w1u, w1m, b1, w2, b2):
  blk = 2048
  grid = BATCH // blk
  return pl.pallas_call(
      _mlp_body,
      grid=(grid,),
      in_specs=[
          pl.BlockSpec((blk, ROW), lambda i: (i, 0)),
          pl.BlockSpec((blk, ROW), lambda i: (i, 0)),
          pl.BlockSpec(w1u.shape, lambda i: (0, 0)),
          pl.BlockSpec(w1m.shape, lambda i: (0, 0)),
          pl.BlockSpec(b1.shape, lambda i: (0, 0)),
          pl.BlockSpec(w2.shape, lambda i: (0, 0)),
          pl.BlockSpec(b2.shape, lambda i: (0, 0)),
      ],
      out_specs=pl.BlockSpec((blk, 1), lambda i: (i, 0)),
      out_shape=jax.ShapeDtypeStruct((BATCH, 1), jnp.float32),
  )(u_rows, m_rows, w1u, w1m, b1, w2, b2)


@jax.jit
def kernel(user_emb_idx, movie_emb_idx, user_table, movie_table, W1, b1, W2, b2):
  u_idx = user_emb_idx.reshape(NW, NCHUNK, CHUNK)
  m_idx = movie_emb_idx.reshape(NW, NCHUNK, CHUNK)
  pad = ((0, 0), (0, ROW - EMBED_DIM))
  u_rows, m_rows = _sc_gather(
      u_idx, m_idx, jnp.pad(user_table, pad), jnp.pad(movie_table, pad))
  w1t = W1.T
  zpad = jnp.zeros((ROW - EMBED_DIM, w1t.shape[1]), w1t.dtype)
  return _mlp(
      u_rows,
      m_rows,
      jnp.concatenate([w1t[:EMBED_DIM], zpad], axis=0),
      jnp.concatenate([w1t[EMBED_DIM:], zpad], axis=0),
      b1.reshape(1, -1),
      W2.T,
      b2.reshape(1, 1),
  )


# speedup vs baseline: 1.3759x; 1.3759x over previous
import functools

import jax
import jax.numpy as jnp
from jax import lax
from jax.experimental import pallas as pl
from jax.experimental.pallas import tpu as pltpu
from jax.experimental.pallas import tpu_sc as plsc

BATCH = 16384
EMBED_DIM = 10
GROUP = 8
GROW = GROUP * EMBED_DIM
NC = 2
NS = 16
NW = NC * NS
B_PER_W = BATCH // NW
CHUNK = 128
NCHUNK = B_PER_W // CHUNK
L = 16


def _gather_body(ur_idx, uoff, mr_idx, moff, ut, mt, u_out, m_out,
                 riu, rim, ofu, ofm, rawu, rawm, outu, outm, sem):
  wid = lax.axis_index("s") * NC + lax.axis_index("c")
  base = wid * B_PER_W
  pltpu.sync_copy(ur_idx.at[wid], riu)
  pltpu.sync_copy(mr_idx.at[wid], rim)
  pltpu.sync_copy(uoff.at[wid], ofu)
  pltpu.sync_copy(moff.at[wid], ofm)
  copies = []
  for j in range(NCHUNK):
    sl = pl.ds(j * CHUNK, CHUNK)
    copies.append(pltpu.async_copy(ut.at[riu.at[j]], rawu.at[sl], sem))
    copies.append(pltpu.async_copy(mt.at[rim.at[j]], rawm.at[sl], sem))
  for c in copies:
    c.wait()
  def step(g, _):
    rows = jax.lax.iota(jnp.int32, L) + g * L
    sl16 = pl.ds(g * L, L)
    offu16 = ofu[sl16]
    offm16 = ofm[sl16]
    for c in range(EMBED_DIM):
      outu[c, sl16] = plsc.load_gather(rawu, [rows, offu16 + c])
      outm[c, sl16] = plsc.load_gather(rawm, [rows, offm16 + c])
    return ()
  lax.fori_loop(0, B_PER_W // L, step, (), unroll=4)
  pltpu.sync_copy(outu, u_out.at[:, pl.ds(base, B_PER_W)])
  pltpu.sync_copy(outm, m_out.at[:, pl.ds(base, B_PER_W)])


_sc_gather = functools.partial(
    pl.kernel,
    out_type=(
        jax.ShapeDtypeStruct((EMBED_DIM, BATCH), jnp.float32),
        jax.ShapeDtypeStruct((EMBED_DIM, BATCH), jnp.float32),
    ),
    mesh=plsc.VectorSubcoreMesh(core_axis_name="c", subcore_axis_name="s"),
    scratch_types=[
        pltpu.VMEM((NCHUNK, CHUNK), jnp.int32),
        pltpu.VMEM((NCHUNK, CHUNK), jnp.int32),
        pltpu.VMEM((B_PER_W,), jnp.int32),
        pltpu.VMEM((B_PER_W,), jnp.int32),
        pltpu.VMEM((B_PER_W, GROW), jnp.float32),
        pltpu.VMEM((B_PER_W, GROW), jnp.float32),
        pltpu.VMEM((EMBED_DIM, B_PER_W), jnp.float32),
        pltpu.VMEM((EMBED_DIM, B_PER_W), jnp.float32),
        pltpu.SemaphoreType.DMA,
    ],
    compiler_params=pltpu.CompilerParams(
        use_tc_tiling_on_sc=False, needs_layout_passes=False),
)(_gather_body)


def _mlp_body(u_ref, m_ref, w1u_ref, w1m_ref, b1_ref, w2_ref, b2_ref, o_ref):
  h = (
      jnp.dot(w1u_ref[...], u_ref[...], preferred_element_type=jnp.float32)
      + jnp.dot(w1m_ref[...], m_ref[...], preferred_element_type=jnp.float32)
      + b1_ref[...]
  )
  h = jnp.maximum(h, 0.0)
  o_ref[...] = (
      jnp.dot(w2_ref[...], h, preferred_element_type=jnp.float32) + b2_ref[...]
  )


def _mlp(u_rows, m_rows, w1u, w1m, b1, w2, b2):
  blk = 2048
  grid = BATCH // blk
  return pl.pallas_call(
      _mlp_body,
      grid=(grid,),
      in_specs=[
          pl.BlockSpec((EMBED_DIM, blk), lambda i: (0, i)),
          pl.BlockSpec((EMBED_DIM, blk), lambda i: (0, i)),
          pl.BlockSpec(w1u.shape, lambda i: (0, 0)),
          pl.BlockSpec(w1m.shape, lambda i: (0, 0)),
          pl.BlockSpec(b1.shape, lambda i: (0, 0)),
          pl.BlockSpec(w2.shape, lambda i: (0, 0)),
          pl.BlockSpec(b2.shape, lambda i: (0, 0)),
      ],
      out_specs=pl.BlockSpec((1, blk), lambda i: (0, i)),
      out_shape=jax.ShapeDtypeStruct((1, BATCH), jnp.float32),
  )(u_rows, m_rows, w1u, w1m, b1, w2, b2)


@jax.jit
def kernel(user_emb_idx, movie_emb_idx, user_table, movie_table, W1, b1, W2, b2):
  u_idx = user_emb_idx.reshape(BATCH)
  m_idx = movie_emb_idx.reshape(BATCH)
  ur = (u_idx >> 3).reshape(NW, NCHUNK, CHUNK)
  mr = (m_idx >> 3).reshape(NW, NCHUNK, CHUNK)
  uoff = ((u_idx & 7) * EMBED_DIM).reshape(NW, B_PER_W)
  moff = ((m_idx & 7) * EMBED_DIM).reshape(NW, B_PER_W)
  ut = user_table.reshape(-1, GROW)
  mt = movie_table.reshape(-1, GROW)
  uT, mT = _sc_gather(ur, uoff, mr, moff, ut, mt)
  out = _mlp(
      uT,
      mT,
      W1[:, :EMBED_DIM],
      W1[:, EMBED_DIM:],
      b1.reshape(-1, 1),
      W2,
      b2.reshape(1, 1),
  )
  return out.reshape(BATCH, 1)


# speedup vs baseline: 8.9486x; 6.5039x over previous
import functools

import jax
import jax.numpy as jnp
from jax import lax
from jax.experimental import pallas as pl
from jax.experimental.pallas import tpu as pltpu
from jax.experimental.pallas import tpu_sc as plsc

BATCH = 16384
EMBED_DIM = 10
NC = 2
NS = 16
NW = NC * NS
B_PER_W = BATCH // NW
CHUNK = 128
NCHUNK = B_PER_W // CHUNK
BN = 32768
BLK_WORDS = EMBED_DIM * BN


def _detile_body(t_ref, o_ref):
  o_ref[...] = t_ref[...].reshape(-1)


def _detile(tT, nb):
  return pl.pallas_call(
      _detile_body,
      grid=(nb,),
      in_specs=[pl.BlockSpec((EMBED_DIM, BN), lambda j: (0, j))],
      out_specs=pl.BlockSpec((BLK_WORDS,), lambda j: (j,)),
      out_shape=jax.ShapeDtypeStruct((nb * BLK_WORDS,), jnp.float32),
  )(tT)


def _gather_body(uoffs, moffs, ufl, mfl, u_out, m_out, offu, offm, outu, outm,
                 sem):
  wid = lax.axis_index("s") * NC + lax.axis_index("c")
  base = wid * B_PER_W
  pltpu.sync_copy(uoffs.at[wid], offu)
  pltpu.sync_copy(moffs.at[wid], offm)
  copies = []
  for j in range(NCHUNK):
    sl = pl.ds(j * CHUNK, CHUNK)
    for c in range(EMBED_DIM):
      row = j * EMBED_DIM + c
      copies.append(pltpu.async_copy(ufl.at[offu.at[row]], outu.at[c, sl], sem))
      copies.append(pltpu.async_copy(mfl.at[offm.at[row]], outm.at[c, sl], sem))
  for cp in copies:
    cp.wait()
  pltpu.sync_copy(outu, u_out.at[:, pl.ds(base, B_PER_W)])
  pltpu.sync_copy(outm, m_out.at[:, pl.ds(base, B_PER_W)])


_sc_gather = functools.partial(
    pl.kernel,
    out_type=(
        jax.ShapeDtypeStruct((EMBED_DIM, BATCH), jnp.float32),
        jax.ShapeDtypeStruct((EMBED_DIM, BATCH), jnp.float32),
    ),
    mesh=plsc.VectorSubcoreMesh(core_axis_name="c", subcore_axis_name="s"),
    scratch_types=[
        pltpu.VMEM((NCHUNK * EMBED_DIM, CHUNK), jnp.int32),
        pltpu.VMEM((NCHUNK * EMBED_DIM, CHUNK), jnp.int32),
        pltpu.VMEM((EMBED_DIM, B_PER_W), jnp.float32),
        pltpu.VMEM((EMBED_DIM, B_PER_W), jnp.float32),
        pltpu.SemaphoreType.DMA,
    ],
    compiler_params=pltpu.CompilerParams(
        use_tc_tiling_on_sc=False, needs_layout_passes=False),
)(_gather_body)


def _flat_offsets(idx):
  jb = idx >> 15
  base = jb * BLK_WORDS + (idx & (BN - 1))
  cols = (jnp.arange(EMBED_DIM, dtype=jnp.int32) * BN)[None, :]
  o = base[:, None] + cols
  o = o.reshape(NW, NCHUNK, CHUNK, EMBED_DIM)
  return o.transpose(0, 1, 3, 2).reshape(NW, NCHUNK * EMBED_DIM, CHUNK)


def _mlp_body(u_ref, m_ref, w1u_ref, w1m_ref, b1_ref, w2_ref, b2_ref, o_ref):
  h = (
      jnp.dot(w1u_ref[...], u_ref[...], preferred_element_type=jnp.float32)
      + jnp.dot(w1m_ref[...], m_ref[...], preferred_element_type=jnp.float32)
      + b1_ref[...]
  )
  h = jnp.maximum(h, 0.0)
  o_ref[...] = (
      jnp.dot(w2_ref[...], h, preferred_element_type=jnp.float32) + b2_ref[...]
  )


def _mlp(u_rows, m_rows, w1u, w1m, b1, w2, b2):
  blk = 2048
  grid = BATCH // blk
  return pl.pallas_call(
      _mlp_body,
      grid=(grid,),
      in_specs=[
          pl.BlockSpec((EMBED_DIM, blk), lambda i: (0, i)),
          pl.BlockSpec((EMBED_DIM, blk), lambda i: (0, i)),
          pl.BlockSpec(w1u.shape, lambda i: (0, 0)),
          pl.BlockSpec(w1m.shape, lambda i: (0, 0)),
          pl.BlockSpec(b1.shape, lambda i: (0, 0)),
          pl.BlockSpec(w2.shape, lambda i: (0, 0)),
          pl.BlockSpec(b2.shape, lambda i: (0, 0)),
      ],
      out_specs=pl.BlockSpec((1, blk), lambda i: (0, i)),
      out_shape=jax.ShapeDtypeStruct((1, BATCH), jnp.float32),
  )(u_rows, m_rows, w1u, w1m, b1, w2, b2)


@jax.jit
def kernel(user_emb_idx, movie_emb_idx, user_table, movie_table, W1, b1, W2, b2):
  nbu = -(-user_table.shape[0] // BN)
  nbm = -(-movie_table.shape[0] // BN)
  ufl = _detile(user_table.T, nbu)
  mfl = _detile(movie_table.T, nbm)
  uoffs = _flat_offsets(user_emb_idx.reshape(BATCH))
  moffs = _flat_offsets(movie_emb_idx.reshape(BATCH))
  uT, mT = _sc_gather(uoffs, moffs, ufl, mfl)
  out = _mlp(
      uT,
      mT,
      W1[:, :EMBED_DIM],
      W1[:, EMBED_DIM:],
      b1.reshape(-1, 1),
      W2,
      b2.reshape(1, 1),
  )
  return out.reshape(BATCH, 1)


# speedup vs baseline: 9.0426x; 1.0105x over previous
import functools

import jax
import jax.numpy as jnp
from jax import lax
from jax.experimental import pallas as pl
from jax.experimental.pallas import tpu as pltpu
from jax.experimental.pallas import tpu_sc as plsc

BATCH = 16384
EMBED_DIM = 10
NC = 2
NS = 16
NW = NC * NS
B_PER_W = BATCH // NW
CHUNK = 128
NCHUNK = B_PER_W // CHUNK
BN = 32768
BLK_WORDS = EMBED_DIM * BN


def _detile_body(t_ref, o_ref):
  o_ref[...] = t_ref[...].reshape(-1)


def _detile(tT, nb):
  return pl.pallas_call(
      _detile_body,
      grid=(nb,),
      in_specs=[pl.BlockSpec((EMBED_DIM, BN), lambda j: (0, j))],
      out_specs=pl.BlockSpec((BLK_WORDS,), lambda j: (j,)),
      out_shape=jax.ShapeDtypeStruct((nb * BLK_WORDS,), jnp.float32),
  )(tT)


def _gather_body(offs, fl, t_out, off, out, sem):
  wid = lax.axis_index("s") * NC + lax.axis_index("c")
  base = wid * B_PER_W
  pltpu.sync_copy(offs.at[wid], off)
  copies = []
  for j in range(NCHUNK):
    sl = pl.ds(j * CHUNK, CHUNK)
    for c in range(EMBED_DIM):
      row = j * EMBED_DIM + c
      copies.append(pltpu.async_copy(fl.at[off.at[row]], out.at[c, sl], sem))
  for cp in copies:
    cp.wait()
  pltpu.sync_copy(out, t_out.at[:, pl.ds(base, B_PER_W)])


_sc_gather = functools.partial(
    pl.kernel,
    out_type=jax.ShapeDtypeStruct((EMBED_DIM, BATCH), jnp.float32),
    mesh=plsc.VectorSubcoreMesh(core_axis_name="c", subcore_axis_name="s"),
    scratch_types=[
        pltpu.VMEM((NCHUNK * EMBED_DIM, CHUNK), jnp.int32),
        pltpu.VMEM((EMBED_DIM, B_PER_W), jnp.float32),
        pltpu.SemaphoreType.DMA,
    ],
    compiler_params=pltpu.CompilerParams(
        use_tc_tiling_on_sc=False, needs_layout_passes=False),
)(_gather_body)


def _flat_offsets(idx):
  jb = idx >> 15
  base = jb * BLK_WORDS + (idx & (BN - 1))
  cols = (jnp.arange(EMBED_DIM, dtype=jnp.int32) * BN)[None, :]
  o = base[:, None] + cols
  o = o.reshape(NW, NCHUNK, CHUNK, EMBED_DIM)
  return o.transpose(0, 1, 3, 2).reshape(NW, NCHUNK * EMBED_DIM, CHUNK)


def _mlp_body(u_ref, m_ref, w1u_ref, w1m_ref, b1_ref, w2_ref, b2_ref, o_ref):
  h = (
      jnp.dot(w1u_ref[...], u_ref[...], preferred_element_type=jnp.float32)
      + jnp.dot(w1m_ref[...], m_ref[...], preferred_element_type=jnp.float32)
      + b1_ref[...]
  )
  h = jnp.maximum(h, 0.0)
  o_ref[...] = (
      jnp.dot(w2_ref[...], h, preferred_element_type=jnp.float32) + b2_ref[...]
  )


def _mlp(u_rows, m_rows, w1u, w1m, b1, w2, b2):
  blk = 2048
  grid = BATCH // blk
  return pl.pallas_call(
      _mlp_body,
      grid=(grid,),
      in_specs=[
          pl.BlockSpec((EMBED_DIM, blk), lambda i: (0, i)),
          pl.BlockSpec((EMBED_DIM, blk), lambda i: (0, i)),
          pl.BlockSpec(w1u.shape, lambda i: (0, 0)),
          pl.BlockSpec(w1m.shape, lambda i: (0, 0)),
          pl.BlockSpec(b1.shape, lambda i: (0, 0)),
          pl.BlockSpec(w2.shape, lambda i: (0, 0)),
          pl.BlockSpec(b2.shape, lambda i: (0, 0)),
      ],
      out_specs=pl.BlockSpec((1, blk), lambda i: (0, i)),
      out_shape=jax.ShapeDtypeStruct((1, BATCH), jnp.float32),
  )(u_rows, m_rows, w1u, w1m, b1, w2, b2)


@jax.jit
def kernel(user_emb_idx, movie_emb_idx, user_table, movie_table, W1, b1, W2, b2):
  nbu = -(-user_table.shape[0] // BN)
  nbm = -(-movie_table.shape[0] // BN)
  uoffs = _flat_offsets(user_emb_idx.reshape(BATCH))
  moffs = _flat_offsets(movie_emb_idx.reshape(BATCH))
  mfl = _detile(movie_table.T, nbm)
  mT = _sc_gather(moffs, mfl)
  ufl = _detile(user_table.T, nbu)
  uT = _sc_gather(uoffs, ufl)
  out = _mlp(
      uT,
      mT,
      W1[:, :EMBED_DIM],
      W1[:, EMBED_DIM:],
      b1.reshape(-1, 1),
      W2,
      b2.reshape(1, 1),
  )
  return out.reshape(BATCH, 1)


# speedup vs baseline: 9.5412x; 1.0551x over previous
import functools

import jax
import jax.numpy as jnp
from jax import lax
from jax.experimental import pallas as pl
from jax.experimental.pallas import tpu as pltpu
from jax.experimental.pallas import tpu_sc as plsc

BATCH = 16384
EMBED_DIM = 10
NC = 2
NS = 16
NW = NC * NS
B_PER_W = BATCH // NW
CHUNK = 128
NCHUNK = B_PER_W // CHUNK
BN = 32768


def _make_detile(rows, row_block):
  def body(t_ref, o_ref):
    o_ref[...] = t_ref[...].reshape(-1)

  def call(tT, nb):
    return pl.pallas_call(
        body,
        grid=(nb,),
        in_specs=[pl.BlockSpec((rows, BN), lambda j: (row_block, j))],
        out_specs=pl.BlockSpec((rows * BN,), lambda j: (j,)),
        out_shape=jax.ShapeDtypeStruct((nb * rows * BN,), jnp.float32),
    )(tT)

  return call


_detile10 = _make_detile(EMBED_DIM, 0)


def _gather_body(uoffs, moffs, ufl, mfl, u_out, m_out, offu, offm, outu, outm,
                 sem):
  wid = lax.axis_index("s") * NC + lax.axis_index("c")
  base = wid * B_PER_W
  pltpu.sync_copy(uoffs.at[wid], offu)
  pltpu.sync_copy(moffs.at[wid], offm)
  copies = []
  for j in range(NCHUNK):
    sl = pl.ds(j * CHUNK, CHUNK)
    for c in range(EMBED_DIM):
      row = j * EMBED_DIM + c
      copies.append(pltpu.async_copy(ufl.at[offu.at[row]], outu.at[c, sl], sem))
      copies.append(pltpu.async_copy(mfl.at[offm.at[row]], outm.at[c, sl], sem))
  for cp in copies:
    cp.wait()
  for c in range(EMBED_DIM):
    dst = pl.ds(c * BATCH + base, B_PER_W)
    pltpu.sync_copy(outu.at[c], u_out.at[dst])
    pltpu.sync_copy(outm.at[c], m_out.at[dst])


_sc_gather = functools.partial(
    pl.kernel,
    out_type=(
        jax.ShapeDtypeStruct((EMBED_DIM * BATCH,), jnp.float32),
        jax.ShapeDtypeStruct((EMBED_DIM * BATCH,), jnp.float32),
    ),
    mesh=plsc.VectorSubcoreMesh(core_axis_name="c", subcore_axis_name="s"),
    scratch_types=[
        pltpu.VMEM((NCHUNK * EMBED_DIM, CHUNK), jnp.int32),
        pltpu.VMEM((NCHUNK * EMBED_DIM, CHUNK), jnp.int32),
        pltpu.VMEM((EMBED_DIM, B_PER_W), jnp.float32),
        pltpu.VMEM((EMBED_DIM, B_PER_W), jnp.float32),
        pltpu.SemaphoreType.DMA,
    ],
    compiler_params=pltpu.CompilerParams(
        use_tc_tiling_on_sc=False, needs_layout_passes=False),
)(_gather_body)


def _flat_offsets(idx):
  jb = idx >> 15
  base = jb * (EMBED_DIM * BN) + (idx & (BN - 1))
  cols = (jnp.arange(EMBED_DIM, dtype=jnp.int32) * BN)[None, :]
  o = base[:, None] + cols
  o = o.reshape(NW, NCHUNK, CHUNK, EMBED_DIM)
  return o.transpose(0, 1, 3, 2).reshape(NW, NCHUNK * EMBED_DIM, CHUNK)


def _mlp_body(u_ref, m_ref, w1u_ref, w1m_ref, b1_ref, w2_ref, b2_ref, o_ref):
  u = u_ref[...].reshape(EMBED_DIM, BATCH)
  m = m_ref[...].reshape(EMBED_DIM, BATCH)
  h = (
      jnp.dot(w1u_ref[...], u, preferred_element_type=jnp.float32)
      + jnp.dot(w1m_ref[...], m, preferred_element_type=jnp.float32)
      + b1_ref[...]
  )
  h = jnp.maximum(h, 0.0)
  o_ref[...] = (
      jnp.dot(w2_ref[...], h, preferred_element_type=jnp.float32) + b2_ref[...]
  )


def _mlp(u_flat, m_flat, w1u, w1m, b1, w2, b2):
  return pl.pallas_call(
      _mlp_body,
      out_shape=jax.ShapeDtypeStruct((1, BATCH), jnp.float32),
  )(u_flat, m_flat, w1u, w1m, b1, w2, b2)


@jax.jit
def kernel(user_emb_idx, movie_emb_idx, user_table, movie_table, W1, b1, W2, b2):
  nbu = -(-user_table.shape[0] // BN)
  nbm = -(-movie_table.shape[0] // BN)
  uoffs = _flat_offsets(user_emb_idx.reshape(BATCH))
  moffs = _flat_offsets(movie_emb_idx.reshape(BATCH))
  mfl = _detile10(movie_table.T, nbm)
  ufl = _detile10(user_table.T, nbu)
  u_flat, m_flat = _sc_gather(uoffs, moffs, ufl, mfl)
  out = _mlp(
      u_flat,
      m_flat,
      W1[:, :EMBED_DIM],
      W1[:, EMBED_DIM:],
      b1.reshape(-1, 1),
      W2,
      b2.reshape(1, 1),
  )
  return out.reshape(BATCH, 1)


# speedup vs baseline: 10.2821x; 1.0777x over previous
import functools

import jax
import jax.numpy as jnp
from jax import lax
from jax.experimental import pallas as pl
from jax.experimental.pallas import tpu as pltpu
from jax.experimental.pallas import tpu_sc as plsc

BATCH = 16384
EMBED_DIM = 10
NC = 2
NS = 16
NW = NC * NS
B_PER_W = BATCH // NW
CHUNK = 128
NCHUNK = B_PER_W // CHUNK
BN = 65536


def _make_detile(rows, row_block):
  def body(t_ref, o_ref):
    o_ref[...] = t_ref[...].reshape(-1)

  def call(tT, nb):
    return pl.pallas_call(
        body,
        grid=(nb,),
        in_specs=[pl.BlockSpec((rows, BN), lambda j: (row_block, j))],
        out_specs=pl.BlockSpec((rows * BN,), lambda j: (j,)),
        out_shape=jax.ShapeDtypeStruct((nb * rows * BN,), jnp.float32),
    )(tT)

  return call


_detile10 = _make_detile(EMBED_DIM, 0)


def _gather_body(uoffs, moffs, ufl, mfl, u_out, m_out, offu, offm, outu, outm,
                 sem):
  wid = lax.axis_index("s") * NC + lax.axis_index("c")
  base = wid * B_PER_W
  pltpu.sync_copy(uoffs.at[wid], offu)
  pltpu.sync_copy(moffs.at[wid], offm)
  copies = []
  for j in range(NCHUNK):
    sl = pl.ds(j * CHUNK, CHUNK)
    for c in range(EMBED_DIM):
      row = j * EMBED_DIM + c
      copies.append(pltpu.async_copy(ufl.at[offu.at[row]], outu.at[c, sl], sem))
      copies.append(pltpu.async_copy(mfl.at[offm.at[row]], outm.at[c, sl], sem))
  for cp in copies:
    cp.wait()
  for c in range(EMBED_DIM):
    dst = pl.ds(c * BATCH + base, B_PER_W)
    pltpu.sync_copy(outu.at[c], u_out.at[dst])
    pltpu.sync_copy(outm.at[c], m_out.at[dst])


_sc_gather = functools.partial(
    pl.kernel,
    out_type=(
        jax.ShapeDtypeStruct((EMBED_DIM * BATCH,), jnp.float32),
        jax.ShapeDtypeStruct((EMBED_DIM * BATCH,), jnp.float32),
    ),
    mesh=plsc.VectorSubcoreMesh(core_axis_name="c", subcore_axis_name="s"),
    scratch_types=[
        pltpu.VMEM((NCHUNK * EMBED_DIM, CHUNK), jnp.int32),
        pltpu.VMEM((NCHUNK * EMBED_DIM, CHUNK), jnp.int32),
        pltpu.VMEM((EMBED_DIM, B_PER_W), jnp.float32),
        pltpu.VMEM((EMBED_DIM, B_PER_W), jnp.float32),
        pltpu.SemaphoreType.DMA,
    ],
    compiler_params=pltpu.CompilerParams(
        use_tc_tiling_on_sc=False, needs_layout_passes=False),
)(_gather_body)


def _flat_offsets(idx):
  jb = idx >> 16
  base = jb * (EMBED_DIM * BN) + (idx & (BN - 1))
  cols = (jnp.arange(EMBED_DIM, dtype=jnp.int32) * BN)[None, :]
  o = base[:, None] + cols
  o = o.reshape(NW, NCHUNK, CHUNK, EMBED_DIM)
  return o.transpose(0, 1, 3, 2).reshape(NW, NCHUNK * EMBED_DIM, CHUNK)


def _mlp_body(u_ref, m_ref, w1u_ref, w1m_ref, b1_ref, w2_ref, b2_ref, o_ref):
  u = u_ref[...].reshape(EMBED_DIM, BATCH)
  m = m_ref[...].reshape(EMBED_DIM, BATCH)
  h = (
      jnp.dot(w1u_ref[...], u, preferred_element_type=jnp.float32)
      + jnp.dot(w1m_ref[...], m, preferred_element_type=jnp.float32)
      + b1_ref[...]
  )
  h = jnp.maximum(h, 0.0)
  o_ref[...] = (
      jnp.dot(w2_ref[...], h, preferred_element_type=jnp.float32) + b2_ref[...]
  )


def _mlp(u_flat, m_flat, w1u, w1m, b1, w2, b2):
  return pl.pallas_call(
      _mlp_body,
      out_shape=jax.ShapeDtypeStruct((1, BATCH), jnp.float32),
  )(u_flat, m_flat, w1u, w1m, b1, w2, b2)


@jax.jit
def kernel(user_emb_idx, movie_emb_idx, user_table, movie_table, W1, b1, W2, b2):
  nbu = -(-user_table.shape[0] // BN)
  nbm = -(-movie_table.shape[0] // BN)
  uoffs = _flat_offsets(user_emb_idx.reshape(BATCH))
  moffs = _flat_offsets(movie_emb_idx.reshape(BATCH))
  mfl = _detile10(movie_table.T, nbm)
  ufl = _detile10(user_table.T, nbu)
  u_flat, m_flat = _sc_gather(uoffs, moffs, ufl, mfl)
  out = _mlp(
      u_flat,
      m_flat,
      W1[:, :EMBED_DIM],
      W1[:, EMBED_DIM:],
      b1.reshape(-1, 1),
      W2,
      b2.reshape(1, 1),
  )
  return out.reshape(BATCH, 1)


# speedup vs baseline: 11.5917x; 1.1274x over previous
import functools

import jax
import jax.numpy as jnp
from jax import lax
from jax.experimental import pallas as pl
from jax.experimental.pallas import tpu as pltpu
from jax.experimental.pallas import tpu_sc as plsc

BATCH = 16384
EMBED_DIM = 10
NPAIR = EMBED_DIM // 2
NC = 2
NS = 16
NW = NC * NS
B_PER_W = BATCH // NW
CHUNK = 128
NCHUNK = B_PER_W // CHUNK
BN = 65536


def _detile_body(t_ref, o_ref):
  y = t_ref[...].astype(jnp.bfloat16)
  u = lax.bitcast_convert_type(y, jnp.uint16).astype(jnp.uint32)
  u3 = u.reshape(NPAIR, 2, BN)
  w = u3[:, 0, :] | (u3[:, 1, :] << 16)
  o_ref[...] = w.astype(jnp.int32).reshape(-1)


def _detile(tT, nb):
  return pl.pallas_call(
      _detile_body,
      grid=(nb,),
      in_specs=[pl.BlockSpec((EMBED_DIM, BN), lambda j: (0, j))],
      out_specs=pl.BlockSpec((NPAIR * BN,), lambda j: (j,)),
      out_shape=jax.ShapeDtypeStruct((nb * NPAIR * BN,), jnp.int32),
  )(tT)


def _gather_body(uoffs, moffs, ufl, mfl, u_out, m_out, offu, offm, outu, outm,
                 sem):
  wid = lax.axis_index("s") * NC + lax.axis_index("c")
  base = wid * B_PER_W
  pltpu.sync_copy(uoffs.at[wid], offu)
  pltpu.sync_copy(moffs.at[wid], offm)
  copies = []
  for j in range(NCHUNK):
    sl = pl.ds(j * CHUNK, CHUNK)
    for p in range(NPAIR):
      row = j * NPAIR + p
      copies.append(pltpu.async_copy(ufl.at[offu.at[row]], outu.at[p, sl], sem))
      copies.append(pltpu.async_copy(mfl.at[offm.at[row]], outm.at[p, sl], sem))
  for cp in copies:
    cp.wait()
  for p in range(NPAIR):
    dst = pl.ds(p * BATCH + base, B_PER_W)
    pltpu.sync_copy(outu.at[p], u_out.at[dst])
    pltpu.sync_copy(outm.at[p], m_out.at[dst])


_sc_gather = functools.partial(
    pl.kernel,
    out_type=(
        jax.ShapeDtypeStruct((NPAIR * BATCH,), jnp.int32),
        jax.ShapeDtypeStruct((NPAIR * BATCH,), jnp.int32),
    ),
    mesh=plsc.VectorSubcoreMesh(core_axis_name="c", subcore_axis_name="s"),
    scratch_types=[
        pltpu.VMEM((NCHUNK * NPAIR, CHUNK), jnp.int32),
        pltpu.VMEM((NCHUNK * NPAIR, CHUNK), jnp.int32),
        pltpu.VMEM((NPAIR, B_PER_W), jnp.int32),
        pltpu.VMEM((NPAIR, B_PER_W), jnp.int32),
        pltpu.SemaphoreType.DMA,
    ],
    compiler_params=pltpu.CompilerParams(
        use_tc_tiling_on_sc=False, needs_layout_passes=False),
)(_gather_body)


def _flat_offsets(idx):
  jb = idx >> 16
  base = jb * (NPAIR * BN) + (idx & (BN - 1))
  cols = (jnp.arange(NPAIR, dtype=jnp.int32) * BN)[None, :]
  o = base[:, None] + cols
  o = o.reshape(NW, NCHUNK, CHUNK, NPAIR)
  return o.transpose(0, 1, 3, 2).reshape(NW, NCHUNK * NPAIR, CHUNK)


def _unpack(w):
  w = w.reshape(NPAIR, BATCH)
  lo = lax.bitcast_convert_type((w & 0xFFFF).astype(jnp.uint16), jnp.bfloat16)
  hi = lax.bitcast_convert_type(
      ((w >> 16) & 0xFFFF).astype(jnp.uint16), jnp.bfloat16)
  return jnp.concatenate([lo, hi], axis=0).astype(jnp.float32)


def _mlp_body(u_ref, m_ref, w1u_ref, w1m_ref, b1_ref, w2_ref, b2_ref, o_ref):
  u = _unpack(u_ref[...])
  m = _unpack(m_ref[...])
  h = (
      jnp.dot(w1u_ref[...], u, preferred_element_type=jnp.float32)
      + jnp.dot(w1m_ref[...], m, preferred_element_type=jnp.float32)
      + b1_ref[...]
  )
  h = jnp.maximum(h, 0.0)
  o_ref[...] = (
      jnp.dot(w2_ref[...], h, preferred_element_type=jnp.float32) + b2_ref[...]
  )


def _mlp(u_flat, m_flat, w1u, w1m, b1, w2, b2):
  return pl.pallas_call(
      _mlp_body,
      out_shape=jax.ShapeDtypeStruct((1, BATCH), jnp.float32),
  )(u_flat, m_flat, w1u, w1m, b1, w2, b2)


@jax.jit
def kernel(user_emb_idx, movie_emb_idx, user_table, movie_table, W1, b1, W2, b2):
  nbu = -(-user_table.shape[0] // BN)
  nbm = -(-movie_table.shape[0] // BN)
  uoffs = _flat_offsets(user_emb_idx.reshape(BATCH))
  moffs = _flat_offsets(movie_emb_idx.reshape(BATCH))
  mfl = _detile(movie_table.T, nbm)
  ufl = _detile(user_table.T, nbu)
  u_flat, m_flat = _sc_gather(uoffs, moffs, ufl, mfl)
  parity = jnp.concatenate(
      [jnp.arange(0, EMBED_DIM, 2), jnp.arange(1, EMBED_DIM, 2)])
  w1u = W1[:, :EMBED_DIM][:, parity]
  w1m = W1[:, EMBED_DIM:][:, parity]
  out = _mlp(
      u_flat,
      m_flat,
      w1u,
      w1m,
      b1.reshape(-1, 1),
      W2,
      b2.reshape(1, 1),
  )
  return out.reshape(BATCH, 1)
